# Initial kernel scaffold; baseline (speedup 1.0000x reference)
import jax
import jax.numpy as jnp
from jax.experimental import pallas as pl


def kernel(node_rep, edge_rep, edge_index, W_lift1, g_lift1, b_lift1, W_lift2, g_lift2, b_lift2, W_lvl1, g_lvl1, b_lvl1, W_lvl2a, g_lvl2a, b_lvl2a, W_lvl2b, g_lvl2b, b_lvl2b, eps1, eps2):
    raise NotImplementedError("write your pallas kernel here")


# speedup vs baseline: 2.6041x; 2.6041x over previous
import functools

import jax
import jax.numpy as jnp
from jax import lax
from jax.experimental import pallas as pl
from jax.experimental.pallas import tpu as pltpu
from jax.experimental.pallas import tpu_sc as plsc

_CHUNK = 80


def _gather_add(node_rep, src3d, dst3d):
    n, d = node_rep.shape
    nw, nch, chunk = src3d.shape
    e = nw * nch * chunk
    info = plsc.get_sparse_core_info()
    nc, ns = info.num_cores, info.num_subcores
    per_w = e // nw

    def body(node_hbm, src_hbm, dst_hbm, out_hbm,
             sidx, didx, sbuf, dbuf, sem1, sem2):
        cid = lax.axis_index("c")
        sid = lax.axis_index("s")
        wid = sid * nc + cid
        pltpu.sync_copy(src_hbm.at[wid], sidx)
        pltpu.sync_copy(dst_hbm.at[wid], didx)

        def chunk_body(j, carry):
            cp1 = pltpu.async_copy(node_hbm.at[sidx.at[j]], sbuf, sem1)
            cp2 = pltpu.async_copy(node_hbm.at[didx.at[j]], dbuf, sem2)
            cp1.wait()
            cp2.wait()

            def row_body(i, c2):
                for l in range(d // 16):
                    sl = pl.ds(l * 16, 16)
                    sbuf[i, sl] = sbuf[i, sl] + dbuf[i, sl]
                return c2

            lax.fori_loop(0, chunk, row_body, 0)
            pltpu.sync_copy(sbuf, out_hbm.at[pl.ds(wid * per_w + j * chunk, chunk)])
            return carry

        lax.fori_loop(0, nch, chunk_body, 0)

    mesh = plsc.VectorSubcoreMesh(core_axis_name="c", subcore_axis_name="s")
    kern = pl.kernel(
        body,
        out_type=jax.ShapeDtypeStruct((e, d), jnp.float32),
        mesh=mesh,
        scratch_types=[
            pltpu.VMEM((nch, chunk), jnp.int32),
            pltpu.VMEM((nch, chunk), jnp.int32),
            pltpu.VMEM((chunk, d), jnp.float32),
            pltpu.VMEM((chunk, d), jnp.float32),
            pltpu.SemaphoreType.DMA,
            pltpu.SemaphoreType.DMA,
        ],
    )
    return kern(node_rep, src3d, dst3d)


def _scatter_add(u1, src3d, dst3d, npad):
    e, d = u1.shape
    ns, nch, chunk = src3d.shape
    info = plsc.get_sparse_core_info()
    nc = info.num_cores
    per_t = e // ns
    half = npad // nc
    rpt = half // ns
    zrows = jnp.zeros((rpt, d), jnp.float32)

    def body(u1_hbm, src_hbm, dst_hbm, z_hbm, out_hbm,
             sidx, didx, dbuf, acc):
        cid = lax.axis_index("c")
        sid = lax.axis_index("s")
        base = cid * half
        pltpu.sync_copy(z_hbm, acc.at[pl.ds(sid * rpt, rpt)])

        @pl.when(sid == 0)
        def _():
            pltpu.sync_copy(z_hbm.at[pl.ds(0, 8)], acc.at[pl.ds(half, 8)])

        pltpu.sync_copy(src_hbm.at[sid], sidx)
        pltpu.sync_copy(dst_hbm.at[sid], didx)

        def fix_body(j, carry):
            for ref in (sidx, didx):
                for l in range(chunk // 16):
                    sl = pl.ds(l * 16, 16)
                    v = ref[j, sl] - base
                    ok = (v >= 0) & (v < half)
                    ref[j, sl] = jnp.where(ok, v, half)
            return carry

        lax.fori_loop(0, nch, fix_body, 0)
        plsc.subcore_barrier()

        def chunk_body(j, carry):
            pltpu.sync_copy(u1_hbm.at[pl.ds(sid * per_t + j * chunk, chunk)], dbuf)
            pltpu.sync_copy(dbuf, acc.at[sidx.at[j]], add=True)
            pltpu.sync_copy(dbuf, acc.at[didx.at[j]], add=True)
            return carry

        lax.fori_loop(0, nch, chunk_body, 0)
        plsc.subcore_barrier()
        pltpu.sync_copy(acc.at[pl.ds(sid * rpt, rpt)],
                        out_hbm.at[pl.ds(base + sid * rpt, rpt)])

    mesh = plsc.VectorSubcoreMesh(core_axis_name="c", subcore_axis_name="s")
    kern = pl.kernel(
        body,
        out_type=jax.ShapeDtypeStruct((npad, d), jnp.float32),
        mesh=mesh,
        scratch_types=[
            pltpu.VMEM((nch, chunk), jnp.int32),
            pltpu.VMEM((nch, chunk), jnp.int32),
            pltpu.VMEM((chunk, d), jnp.float32),
            pltpu.VMEM_SHARED((half + 8, d), jnp.float32),
        ],
    )
    return kern(u1, src3d, dst3d, zrows)


def _stage1_body(eps2_ref, a_ref, r_ref, w1a_ref, w1b_ref, wl1_ref,
                 y1_ref, eh_ref, s1_ref, s2_ref):
    i = pl.program_id(0)
    a = a_ref[...]
    r = r_ref[...]
    y1 = (jnp.dot(a, w1a_ref[...], preferred_element_type=jnp.float32)
          + jnp.dot(r, w1b_ref[...], preferred_element_type=jnp.float32))
    eh = a + (1.0 + eps2_ref[0, 0]) * r
    y2 = jnp.dot(eh, wl1_ref[...], preferred_element_type=jnp.float32)
    y1_ref[...] = y1
    eh_ref[...] = eh
    s1 = jnp.concatenate([jnp.sum(y1, 0, keepdims=True),
                          jnp.sum(y1 * y1, 0, keepdims=True)], 0)
    s2 = jnp.concatenate([jnp.sum(y2, 0, keepdims=True),
                          jnp.sum(y2 * y2, 0, keepdims=True)], 0)

    @pl.when(i == 0)
    def _():
        s1_ref[...] = jnp.zeros_like(s1_ref)
        s2_ref[...] = jnp.zeros_like(s2_ref)

    s1_ref[...] += s1
    s2_ref[...] += s2


def _stage1(a, r, w1a, w1b, wl1, eps2, te):
    e, d = a.shape
    m = wl1.shape[1]
    return pl.pallas_call(
        _stage1_body,
        grid=(e // te,),
        in_specs=[
            pl.BlockSpec((1, 1), lambda i: (0, 0)),
            pl.BlockSpec((te, d), lambda i: (i, 0)),
            pl.BlockSpec((te, d), lambda i: (i, 0)),
            pl.BlockSpec((d, d), lambda i: (0, 0)),
            pl.BlockSpec((d, d), lambda i: (0, 0)),
            pl.BlockSpec((d, m), lambda i: (0, 0)),
        ],
        out_specs=[
            pl.BlockSpec((te, d), lambda i: (i, 0)),
            pl.BlockSpec((te, d), lambda i: (i, 0)),
            pl.BlockSpec((2, d), lambda i: (0, 0)),
            pl.BlockSpec((2, m), lambda i: (0, 0)),
        ],
        out_shape=[
            jax.ShapeDtypeStruct((e, d), jnp.float32),
            jax.ShapeDtypeStruct((e, d), jnp.float32),
            jax.ShapeDtypeStruct((2, d), jnp.float32),
            jax.ShapeDtypeStruct((2, m), jnp.float32),
        ],
    )(eps2, a, r, w1a, w1b, wl1)


def _bn_coeffs(s_ref, g_ref, b_ref, inv_e):
    s = s_ref[...]
    mean = s[0:1] * inv_e
    var = s[1:2] * inv_e - mean * mean
    scale = g_ref[...] * lax.rsqrt(var + 1e-5)
    shift = b_ref[...] - mean * scale
    return scale, shift


def _stage2_body(y1_ref, eh_ref, s1_ref, s2_ref, wl1_ref, wl2_ref,
                 g1_ref, b1_ref, g2_ref, b2_ref,
                 u1_ref, y3_ref, s3_ref, *, inv_e):
    i = pl.program_id(0)
    sc1, sh1 = _bn_coeffs(s1_ref, g1_ref, b1_ref, inv_e)
    u1_ref[...] = jnp.maximum(y1_ref[...] * sc1 + sh1, 0.0)
    sc2, sh2 = _bn_coeffs(s2_ref, g2_ref, b2_ref, inv_e)
    y2 = jnp.dot(eh_ref[...], wl1_ref[...], preferred_element_type=jnp.float32)
    u2 = jnp.maximum(y2 * sc2 + sh2, 0.0)
    y3 = jnp.dot(u2, wl2_ref[...], preferred_element_type=jnp.float32)
    y3_ref[...] = y3
    s3 = jnp.concatenate([jnp.sum(y3, 0, keepdims=True),
                          jnp.sum(y3 * y3, 0, keepdims=True)], 0)

    @pl.when(i == 0)
    def _():
        s3_ref[...] = jnp.zeros_like(s3_ref)

    s3_ref[...] += s3


def _stage2(y1, eh, s1, s2, wl1, wl2, g1, b1, g2, b2, te):
    e, d = y1.shape
    m = wl1.shape[1]
    return pl.pallas_call(
        functools.partial(_stage2_body, inv_e=1.0 / e),
        grid=(e // te,),
        in_specs=[
            pl.BlockSpec((te, d), lambda i: (i, 0)),
            pl.BlockSpec((te, d), lambda i: (i, 0)),
            pl.BlockSpec((2, d), lambda i: (0, 0)),
            pl.BlockSpec((2, m), lambda i: (0, 0)),
            pl.BlockSpec((d, m), lambda i: (0, 0)),
            pl.BlockSpec((m, d), lambda i: (0, 0)),
            pl.BlockSpec((1, d), lambda i: (0, 0)),
            pl.BlockSpec((1, d), lambda i: (0, 0)),
            pl.BlockSpec((1, m), lambda i: (0, 0)),
            pl.BlockSpec((1, m), lambda i: (0, 0)),
        ],
        out_specs=[
            pl.BlockSpec((te, d), lambda i: (i, 0)),
            pl.BlockSpec((te, d), lambda i: (i, 0)),
            pl.BlockSpec((2, d), lambda i: (0, 0)),
        ],
        out_shape=[
            jax.ShapeDtypeStruct((e, d), jnp.float32),
            jax.ShapeDtypeStruct((e, d), jnp.float32),
            jax.ShapeDtypeStruct((2, d), jnp.float32),
        ],
    )(y1, eh, s1, s2, wl1, wl2, g1, b1, g2, b2)


def _finalize_body(y_ref, s_ref, g_ref, b_ref, o_ref, *, inv_e):
    sc, sh = _bn_coeffs(s_ref, g_ref, b_ref, inv_e)
    o_ref[...] = jnp.maximum(y_ref[...] * sc + sh, 0.0)


def _finalize(y3, s3, g, b, te):
    e, d = y3.shape
    return pl.pallas_call(
        functools.partial(_finalize_body, inv_e=1.0 / e),
        grid=(e // te,),
        in_specs=[
            pl.BlockSpec((te, d), lambda i: (i, 0)),
            pl.BlockSpec((2, d), lambda i: (0, 0)),
            pl.BlockSpec((1, d), lambda i: (0, 0)),
            pl.BlockSpec((1, d), lambda i: (0, 0)),
        ],
        out_specs=pl.BlockSpec((te, d), lambda i: (i, 0)),
        out_shape=jax.ShapeDtypeStruct((e, d), jnp.float32),
    )(y3, s3, g, b)


def _node_body(eps1_ref, node_ref, p_ref, w2a_ref, g2a_ref, b2a_ref,
               w2b_ref, g2b_ref, b2b_ref, o_ref, *, inv_n):
    n = node_ref.shape[0]
    lvl = p_ref[:n]
    nh = (1.0 + eps1_ref[0, 0]) * node_ref[...] + lvl
    y4 = jnp.dot(nh, w2a_ref[...], preferred_element_type=jnp.float32)
    m4 = jnp.sum(y4, 0, keepdims=True) * inv_n
    v4 = jnp.sum(y4 * y4, 0, keepdims=True) * inv_n - m4 * m4
    u4 = jnp.maximum((y4 - m4) * lax.rsqrt(v4 + 1e-5) * g2a_ref[...]
                     + b2a_ref[...], 0.0)
    y5 = jnp.dot(u4, w2b_ref[...], preferred_element_type=jnp.float32)
    m5 = jnp.sum(y5, 0, keepdims=True) * inv_n
    v5 = jnp.sum(y5 * y5, 0, keepdims=True) * inv_n - m5 * m5
    o_ref[...] = jnp.maximum((y5 - m5) * lax.rsqrt(v5 + 1e-5) * g2b_ref[...]
                             + b2b_ref[...], 0.0)


def _node_branch(node_rep, p, w2a, g2a, b2a, w2b, g2b, b2b, eps1):
    n, d = node_rep.shape
    return pl.pallas_call(
        functools.partial(_node_body, inv_n=1.0 / n),
        out_shape=jax.ShapeDtypeStruct((n, d), jnp.float32),
    )(eps1, node_rep, p, w2a, g2a, b2a, w2b, g2b, b2b)


def kernel(node_rep, edge_rep, edge_index,
           W_lift1, g_lift1, b_lift1, W_lift2, g_lift2, b_lift2,
           W_lvl1, g_lvl1, b_lvl1,
           W_lvl2a, g_lvl2a, b_lvl2a, W_lvl2b, g_lvl2b, b_lvl2b,
           eps1, eps2):
    n, d = node_rep.shape
    e = edge_rep.shape[0]
    m = W_lift1.shape[1]
    te = 2000

    info = plsc.get_sparse_core_info()
    nc, ns = info.num_cores, info.num_subcores
    nw = nc * ns
    unit = nc * ns * 8
    npad = ((n + unit - 1) // unit) * unit

    src3d = edge_index[0].reshape(nw, (e // nw) // _CHUNK, _CHUNK)
    dst3d = edge_index[1].reshape(nw, (e // nw) // _CHUNK, _CHUNK)
    src3s = edge_index[0].reshape(ns, (e // ns) // _CHUNK, _CHUNK)
    dst3s = edge_index[1].reshape(ns, (e // ns) // _CHUNK, _CHUNK)
    eps1v = jnp.reshape(eps1, (1, 1))
    eps2v = jnp.reshape(eps2, (1, 1))

    a = _gather_add(node_rep, src3d, dst3d)

    y1, eh, s1, s2 = _stage1(a, edge_rep, W_lvl1[:d], W_lvl1[d:],
                             W_lift1, eps2v, te)
    u1, y3, s3 = _stage2(y1, eh, s1, s2, W_lift1, W_lift2,
                         g_lvl1.reshape(1, d), b_lvl1.reshape(1, d),
                         g_lift1.reshape(1, m), b_lift1.reshape(1, m), te)

    p = _scatter_add(u1, src3s, dst3s, npad)

    node_out = _node_branch(node_rep, p, W_lvl2a,
                            g_lvl2a.reshape(1, m), b_lvl2a.reshape(1, m),
                            W_lvl2b, g_lvl2b.reshape(1, d), b_lvl2b.reshape(1, d),
                            eps1v)
    edge_out = _finalize(y3, s3, g_lift2.reshape(1, d), b_lift2.reshape(1, d), te)
    return (node_out, edge_out)
